# baseline (speedup 1.0000x reference)
import functools

import jax
import jax.numpy as jnp
from jax import lax
from jax.experimental import pallas as pl
from jax.experimental.pallas import tpu as pltpu
from jax.experimental.pallas import tpu_sc as plsc

_V = 1000000
_D = 16
_F = 26
_H = 100
_B = 4096

_NC = 2
_NS = 16
_NW = _NC * _NS
_SPW = _B // _NW
_XR = _F * _D + _F


@functools.lru_cache(maxsize=None)
def _make_sc_gather():
    mesh = plsc.VectorSubcoreMesh(core_axis_name="c", subcore_axis_name="s")

    @functools.partial(
        pl.kernel,
        out_type=jax.ShapeDtypeStruct((_XR, _B), jnp.float32),
        mesh=mesh,
        compiler_params=pltpu.CompilerParams(needs_layout_passes=False),
        scratch_types=[
            pltpu.VMEM((_F, _SPW), jnp.int32),
            pltpu.VMEM((_XR, _SPW), jnp.float32),
            pltpu.SemaphoreType.DMA,
        ],
    )
    def sc_gather(idxt_hbm, tdt_hbm, widet_hbm, out_hbm, idx_v, ext_v, sem):
        wid = lax.axis_index("s") * _NC + lax.axis_index("c")
        base = wid * _SPW
        pltpu.sync_copy(idxt_hbm.at[:, pl.ds(base, _SPW)], idx_v)

        @pl.loop(0, _F)
        def _fire(f):
            ivec = idx_v.at[f]
            for d in range(_D):
                pltpu.async_copy(tdt_hbm.at[pl.ds(d * _V, _V)].at[ivec],
                                 ext_v.at[f * _D + d], sem)
            pltpu.async_copy(widet_hbm.at[ivec],
                             ext_v.at[_F * _D + f], sem)

        @pl.loop(0, _F)
        def _drain(f):
            for _ in range(_D + 1):
                pltpu.make_async_copy(
                    tdt_hbm.at[pl.ds(0, _SPW)], ext_v.at[0], sem
                ).wait()

        pltpu.sync_copy(ext_v, out_hbm.at[:, pl.ds(base, _SPW)])

    return sc_gather


_BB = 512


def _mlp_body(xt_ref, w1_ref, b1_ref, w2_ref, b2_ref, w3_ref, b3_ref,
              wp_ref, bp_ref, o_ref):
    xt = xt_ref[...]
    xd = xt[: _F * _D, :]
    wide = jnp.sum(xt[_F * _D:, :], axis=0, keepdims=True)
    h = jnp.maximum(jnp.dot(w1_ref[...], xd, preferred_element_type=jnp.float32)
                    + b1_ref[...], 0.0)
    h = jnp.maximum(jnp.dot(w2_ref[...], h, preferred_element_type=jnp.float32)
                    + b2_ref[...], 0.0)
    h = jnp.maximum(jnp.dot(w3_ref[...], h, preferred_element_type=jnp.float32)
                    + b3_ref[...], 0.0)
    logits = (jnp.dot(wp_ref[...], h, preferred_element_type=jnp.float32)
              + bp_ref[...] + wide)
    o_ref[...] = jax.nn.sigmoid(logits)


@jax.jit
def _mlp(xt, W1t, b1, W2t, b2, W3t, b3, Wpt, bp):
    grid = (_B // _BB,)
    return pl.pallas_call(
        _mlp_body,
        grid=grid,
        in_specs=[
            pl.BlockSpec((_XR, _BB), lambda i: (0, i)),
            pl.BlockSpec((_H, _F * _D), lambda i: (0, 0)),
            pl.BlockSpec((_H, 1), lambda i: (0, 0)),
            pl.BlockSpec((_H, _H), lambda i: (0, 0)),
            pl.BlockSpec((_H, 1), lambda i: (0, 0)),
            pl.BlockSpec((_H, _H), lambda i: (0, 0)),
            pl.BlockSpec((_H, 1), lambda i: (0, 0)),
            pl.BlockSpec((1, _H), lambda i: (0, 0)),
            pl.BlockSpec((1, 1), lambda i: (0, 0)),
        ],
        out_specs=pl.BlockSpec((1, _BB), lambda i: (0, i)),
        out_shape=jax.ShapeDtypeStruct((1, _B), jnp.float32),
    )(xt, W1t, b1, W2t, b2, W3t, b3, Wpt, bp)


def kernel(inputs, dense_emb, wide_emb, W1, b1, W2, b2, W3, b3, Wp, bp):
    xt = _make_sc_gather()(inputs.T, dense_emb.T.reshape(-1),
                           wide_emb.T.reshape(-1))
    out = _mlp(xt, W1.T, b1.reshape(_H, 1), W2.T, b2.reshape(_H, 1),
               W3.T, b3.reshape(_H, 1), Wp.T, bp.reshape(1, 1))
    return out.reshape(_B, 1)
